# Initial kernel scaffold; baseline (speedup 1.0000x reference)
import jax
import jax.numpy as jnp
from jax.experimental import pallas as pl


def kernel(x, den_src, den_dst, den_pdf, den_tprobs, den_init, den_final, num_src, num_dst, num_pdf, num_tprobs, num_init, num_final):
    raise NotImplementedError("write your pallas kernel here")


# speedup vs baseline: 10.2913x; 10.2913x over previous
import functools

import jax
import jax.numpy as jnp
from jax import lax
from jax.experimental import pallas as pl
from jax.experimental.pallas import tpu as pltpu
from jax.experimental.pallas import tpu_sc as plsc

L = 16
B, T, P = 32, 100, 2048
LEAKY = 1e-05

S_DEN, E_DEN = 1000, 20000
S_NUM, E_NUM = 200, 1000
SP_DEN, SP_NUM = 1024, 208
EP_DEN, EP_NUM = 20000, 1008
OUTW = 128


def _sc_body(x_hbm, dpk_hbm, dtp_hbm, npk_hbm, ntp_hbm,
             dinit_hbm, dfinal_hbm, ninit_hbm, nfinal_hbm,
             dout_hbm, nout_hbm,
             dpk, dtp, npk, ntp,
             dinit, dfinal, dalpha, daleak, dpriv,
             ninit, nfinal, nalpha, naleak, npriv,
             xa, xb, ex, outd, outn,
             sema, semb):
    b = lax.axis_index("s") * 2 + lax.axis_index("c")

    pltpu.async_copy(x_hbm.at[b, 0], xa, sema)
    pltpu.async_copy(x_hbm.at[b, 1], xb, semb)

    pltpu.sync_copy(dpk_hbm, dpk)
    pltpu.sync_copy(dtp_hbm, dtp)
    pltpu.sync_copy(npk_hbm, npk)
    pltpu.sync_copy(ntp_hbm, ntp)
    pltpu.sync_copy(dinit_hbm, dinit)
    pltpu.sync_copy(dfinal_hbm, dfinal)
    pltpu.sync_copy(ninit_hbm, ninit)
    pltpu.sync_copy(nfinal_hbm, nfinal)

    zeros = jnp.zeros((L,), jnp.float32)
    ones = jnp.ones((L,), jnp.float32)
    lanes = lax.iota(jnp.int32, L)

    def init_state(init_r, aleak_r, alpha_r, priv_r, n_sc):
        def bodyf(j, _):
            sl = pl.ds(j * L, L)
            iv = init_r[sl]
            aleak_r[sl] = iv + LEAKY * iv
            alpha_r[sl] = zeros
            for r in range(L):
                priv_r[r, sl] = zeros
            return 0
        lax.fori_loop(0, n_sc, bodyf, 0)

    init_state(dinit, daleak, dalpha, dpriv, SP_DEN // L)
    init_state(ninit, naleak, nalpha, npriv, SP_NUM // L)

    def initout(j, _):
        sl = pl.ds(j * L, L)
        outd[sl] = ones
        outn[sl] = ones
        return 0
    lax.fori_loop(0, OUTW // L, initout, 0)

    def hmm_step(t, pk, tp, init_r, aleak, alpha_r, priv, outr, n_ec, n_sc):
        def edge_body(i, _):
            sl = pl.ds(i * L, L)
            p = pk[sl]
            w = tp[sl]
            srcv = p & 1023
            dstv = lax.shift_right_logical(p, 10) & 1023
            pdfv = lax.shift_right_logical(p, 21)
            a = plsc.load_gather(aleak, [srcv])
            e = plsc.load_gather(ex, [pdfv])
            plsc.addupdate_scatter(priv, [lanes, dstv], a * w * e)
            return 0
        lax.fori_loop(0, n_ec, edge_body, 0)

        def red_body(j, acc):
            sl = pl.ds(j * L, L)
            s = zeros
            for r in range(L):
                s = s + priv[r, sl]
                priv[r, sl] = zeros
            alpha_r[sl] = s
            return acc + s
        sacc = lax.fori_loop(0, n_sc, red_body, zeros)
        scale_v = jnp.full((L,), jnp.sum(sacc) + 1e-30, jnp.float32)
        inv_v = ones / scale_v
        plsc.store_scatter(outr, [jnp.full((L,), t, jnp.int32)], scale_v,
                           mask=lanes == 0)

        def p2(j, _):
            sl = pl.ds(j * L, L)
            aleak[sl] = alpha_r[sl] * inv_v + init_r[sl] * LEAKY
            return 0
        lax.fori_loop(0, n_sc, p2, 0)
        return inv_v

    def compute_ex(xbuf):
        def bodyf(k, _):
            sl = pl.ds(k * L, L)
            ex[sl] = jnp.exp(jnp.clip(xbuf[sl], -30.0, 30.0))
            return 0
        lax.fori_loop(0, P // L, bodyf, 0)

    def substep(t, xbuf, sem):
        pltpu.make_async_copy(x_hbm.at[b, 0], xbuf, sem).wait()
        compute_ex(xbuf)

        @pl.when(t + 2 < T)
        def _():
            pltpu.async_copy(x_hbm.at[b, t + 2], xbuf, sem)

        inv_d = hmm_step(t, dpk, dtp, dinit, daleak, dalpha, dpriv, outd,
                         EP_DEN // L, SP_DEN // L)
        inv_n = hmm_step(t, npk, ntp, ninit, naleak, nalpha, npriv, outn,
                         EP_NUM // L, SP_NUM // L)
        return inv_d, inv_n

    def t2_body(t2, carry):
        t0 = 2 * t2
        substep(t0, xa, sema)
        return substep(t0 + 1, xb, semb)

    inv_d, inv_n = lax.fori_loop(0, T // 2, t2_body, (ones, ones))

    def dot_graph(alpha_r, final_r, inv_v, outr, n_sc):
        def bodyf(j, acc):
            sl = pl.ds(j * L, L)
            return acc + alpha_r[sl] * final_r[sl]
        dacc = lax.fori_loop(0, n_sc, bodyf, zeros)
        dot_v = jnp.full((L,), jnp.sum(dacc), jnp.float32) * inv_v + 1e-30
        plsc.store_scatter(outr, [jnp.full((L,), OUTW - 1, jnp.int32)],
                           dot_v, mask=lanes == 0)

    dot_graph(dalpha, dfinal, inv_d, outd, SP_DEN // L)
    dot_graph(nalpha, nfinal, inv_n, outn, SP_NUM // L)

    pltpu.sync_copy(outd, dout_hbm.at[b])
    pltpu.sync_copy(outn, nout_hbm.at[b])


_SC_SCRATCH = [
    pltpu.VMEM((EP_DEN,), jnp.int32),
    pltpu.VMEM((EP_DEN,), jnp.float32),
    pltpu.VMEM((EP_NUM,), jnp.int32),
    pltpu.VMEM((EP_NUM,), jnp.float32),
    pltpu.VMEM((SP_DEN,), jnp.float32),
    pltpu.VMEM((SP_DEN,), jnp.float32),
    pltpu.VMEM((SP_DEN,), jnp.float32),
    pltpu.VMEM((SP_DEN,), jnp.float32),
    pltpu.VMEM((L, SP_DEN), jnp.float32),
    pltpu.VMEM((SP_NUM,), jnp.float32),
    pltpu.VMEM((SP_NUM,), jnp.float32),
    pltpu.VMEM((SP_NUM,), jnp.float32),
    pltpu.VMEM((SP_NUM,), jnp.float32),
    pltpu.VMEM((L, SP_NUM), jnp.float32),
    pltpu.VMEM((P,), jnp.float32),
    pltpu.VMEM((P,), jnp.float32),
    pltpu.VMEM((P,), jnp.float32),
    pltpu.VMEM((OUTW,), jnp.float32),
    pltpu.VMEM((OUTW,), jnp.float32),
    pltpu.SemaphoreType.DMA,
    pltpu.SemaphoreType.DMA,
]

_sc_fwd = functools.partial(
    pl.kernel,
    out_type=(jax.ShapeDtypeStruct((B, OUTW), jnp.float32),
              jax.ShapeDtypeStruct((B, OUTW), jnp.float32)),
    mesh=plsc.VectorSubcoreMesh(core_axis_name="c", subcore_axis_name="s"),
    scratch_types=_SC_SCRATCH,
    compiler_params=pltpu.CompilerParams(needs_layout_passes=False),
)(_sc_body)


def _fin_body(d_ref, n_ref, o_ref):
    o_ref[0, 0] = (jnp.sum(jnp.log(d_ref[...])) -
                   jnp.sum(jnp.log(n_ref[...]))) / float(B * T)


def _finalize(dout, nout):
    return pl.pallas_call(
        _fin_body,
        out_shape=jax.ShapeDtypeStruct((1, 1), jnp.float32),
        in_specs=[pl.BlockSpec(memory_space=pltpu.VMEM),
                  pl.BlockSpec(memory_space=pltpu.VMEM)],
        out_specs=pl.BlockSpec(memory_space=pltpu.SMEM),
    )(dout, nout)[0, 0]


def _pad1d(v, n, dtype):
    out = jnp.zeros((n,), dtype)
    return out.at[: v.shape[0]].set(v.astype(dtype))


def kernel(x, den_src, den_dst, den_pdf, den_tprobs, den_init, den_final,
           num_src, num_dst, num_pdf, num_tprobs, num_init, num_final):
    f32 = jnp.float32
    i32 = jnp.int32
    dpk = (den_src.astype(i32) | (den_dst.astype(i32) << 10)
           | (den_pdf.astype(i32) << 21))
    npk = (num_src.astype(i32) | (num_dst.astype(i32) << 10)
           | (num_pdf.astype(i32) << 21))
    dpk = _pad1d(dpk, EP_DEN, i32)
    dtp = _pad1d(den_tprobs, EP_DEN, f32)
    npk = _pad1d(npk, EP_NUM, i32)
    ntp = _pad1d(num_tprobs, EP_NUM, f32)
    dinit = _pad1d(den_init, SP_DEN, f32)
    dfinal = _pad1d(den_final, SP_DEN, f32)
    ninit = _pad1d(num_init, SP_NUM, f32)
    nfinal = _pad1d(num_final, SP_NUM, f32)

    dout, nout = _sc_fwd(x.astype(f32), dpk, dtp, npk, ntp,
                         dinit, dfinal, ninit, nfinal)
    return _finalize(dout, nout)


# speedup vs baseline: 37.7356x; 3.6667x over previous
import functools

import jax
import jax.numpy as jnp
from jax import lax
from jax.experimental import pallas as pl
from jax.experimental.pallas import tpu as pltpu
from jax.experimental.pallas import tpu_sc as plsc

L = 16
B, T, P = 32, 100, 2048
LEAKY = 1e-05

S_DEN, E_DEN = 1000, 20000
S_NUM, E_NUM = 200, 1000
SP_DEN, SP_NUM = 1024, 256
EP_DEN, EP_NUM = 20000, 1024
EU_DEN, EU_NUM = 5, 8
SU = 8
OUTW = 128


def _sc_body(x_hbm, dpk_hbm, dtp_hbm, npk_hbm, ntp_hbm,
             dinit_hbm, dfinal_hbm, ninit_hbm, nfinal_hbm,
             dout_hbm, nout_hbm,
             dpk, dtp, npk, ntp,
             dinit, dfinal, dalpha, daleak,
             ninit, nfinal, nalpha, naleak,
             xa, xb, ex, outd, outn,
             sema, semb):
    b = lax.axis_index("s") * 2 + lax.axis_index("c")

    pltpu.async_copy(x_hbm.at[b, 0], xa, sema)
    pltpu.async_copy(x_hbm.at[b, 1], xb, semb)

    pltpu.sync_copy(dpk_hbm, dpk)
    pltpu.sync_copy(dtp_hbm, dtp)
    pltpu.sync_copy(npk_hbm, npk)
    pltpu.sync_copy(ntp_hbm, ntp)
    pltpu.sync_copy(dinit_hbm, dinit)
    pltpu.sync_copy(dfinal_hbm, dfinal)
    pltpu.sync_copy(ninit_hbm, ninit)
    pltpu.sync_copy(nfinal_hbm, nfinal)

    zeros = jnp.zeros((L,), jnp.float32)
    ones = jnp.ones((L,), jnp.float32)
    lanes = lax.iota(jnp.int32, L)

    def init_state(init_r, aleak_r, alpha_r, n_sc):
        def bodyf(j, _):
            sl = pl.ds(j * L, L)
            iv = init_r[sl]
            aleak_r[sl] = iv + LEAKY * iv
            alpha_r[sl] = zeros
            return 0
        lax.fori_loop(0, n_sc, bodyf, 0)

    init_state(dinit, daleak, dalpha, SP_DEN // L)
    init_state(ninit, naleak, nalpha, SP_NUM // L)

    def initout(j, _):
        sl = pl.ds(j * L, L)
        outd[sl] = ones
        outn[sl] = ones
        return 0
    lax.fori_loop(0, OUTW // L, initout, 0)

    def hmm_step(t, pk, tp, init_r, final_r, aleak, alpha_r, outr,
                 n_ec, n_sc, eu):
        @plsc.parallel_loop(0, n_ec * L, step=2 * L, unroll=eu, carry=zeros)
        def edge_body(off, macc):
            wab = tp[pl.ds(lax.shift_right_logical(off, 1), L)]
            wa = plsc.bitcast(lax.shift_left(wab, 16), jnp.float32)
            wb = plsc.bitcast(wab & jnp.int32(-65536), jnp.float32)
            for k, w in ((0, wa), (1, wb)):
                sl = pl.ds(off + k * L, L)
                p = pk[sl]
                srcv = p & 1023
                dstv = lax.shift_right_logical(p, 10) & 1023
                pdfv = lax.shift_right_logical(p, 21)
                a = plsc.load_gather(aleak, [srcv])
                e = plsc.load_gather(ex, [pdfv])
                msg = a * w * e
                plsc.addupdate_scatter(alpha_r, [dstv], msg)
                macc = macc + msg
            return macc

        scale_v = jnp.full((L,), jnp.sum(edge_body) + 1e-30, jnp.float32)
        inv_v = ones / scale_v
        plsc.store_scatter(outr, [jnp.full((L,), t, jnp.int32)], scale_v,
                           mask=lanes == 0)

        @pl.when(t == T - 1)
        def _():
            def dot_body(j, acc):
                sl = pl.ds(j * L, L)
                return acc + alpha_r[sl] * final_r[sl]
            dacc = lax.fori_loop(0, n_sc, dot_body, zeros)
            dot_v = jnp.full((L,), jnp.sum(dacc), jnp.float32) * inv_v + 1e-30
            plsc.store_scatter(outr, [jnp.full((L,), OUTW - 1, jnp.int32)],
                               dot_v, mask=lanes == 0)

        @plsc.parallel_loop(0, n_sc * L, step=L, unroll=SU)
        def p2(off):
            sl = pl.ds(off, L)
            aleak[sl] = alpha_r[sl] * inv_v + init_r[sl] * LEAKY
            alpha_r[sl] = zeros

    def compute_ex(xbuf):
        @plsc.parallel_loop(0, P, step=L, unroll=SU)
        def bodyf(off):
            sl = pl.ds(off, L)
            ex[sl] = jnp.exp(jnp.clip(xbuf[sl], -30.0, 30.0))

    def substep(t, xbuf, sem):
        pltpu.make_async_copy(x_hbm.at[b, 0], xbuf, sem).wait()
        compute_ex(xbuf)

        @pl.when(t + 2 < T)
        def _():
            pltpu.async_copy(x_hbm.at[b, t + 2], xbuf, sem)

        hmm_step(t, dpk, dtp, dinit, dfinal, daleak, dalpha, outd,
                 EP_DEN // L, SP_DEN // L, EU_DEN)
        hmm_step(t, npk, ntp, ninit, nfinal, naleak, nalpha, outn,
                 EP_NUM // L, SP_NUM // L, EU_NUM)

    def t2_body(t2, carry):
        t0 = 2 * t2
        substep(t0, xa, sema)
        substep(t0 + 1, xb, semb)
        return carry

    lax.fori_loop(0, T // 2, t2_body, 0)

    pltpu.sync_copy(outd, dout_hbm.at[b])
    pltpu.sync_copy(outn, nout_hbm.at[b])


_SC_SCRATCH = [
    pltpu.VMEM((EP_DEN,), jnp.int32),
    pltpu.VMEM((EP_DEN // 2,), jnp.int32),
    pltpu.VMEM((EP_NUM,), jnp.int32),
    pltpu.VMEM((EP_NUM // 2,), jnp.int32),
    pltpu.VMEM((SP_DEN,), jnp.float32),
    pltpu.VMEM((SP_DEN,), jnp.float32),
    pltpu.VMEM((SP_DEN,), jnp.float32),
    pltpu.VMEM((SP_DEN,), jnp.float32),
    pltpu.VMEM((SP_NUM,), jnp.float32),
    pltpu.VMEM((SP_NUM,), jnp.float32),
    pltpu.VMEM((SP_NUM,), jnp.float32),
    pltpu.VMEM((SP_NUM,), jnp.float32),
    pltpu.VMEM((P,), jnp.float32),
    pltpu.VMEM((P,), jnp.float32),
    pltpu.VMEM((P,), jnp.float32),
    pltpu.VMEM((OUTW,), jnp.float32),
    pltpu.VMEM((OUTW,), jnp.float32),
    pltpu.SemaphoreType.DMA,
    pltpu.SemaphoreType.DMA,
]

_sc_fwd = functools.partial(
    pl.kernel,
    out_type=(jax.ShapeDtypeStruct((B, OUTW), jnp.float32),
              jax.ShapeDtypeStruct((B, OUTW), jnp.float32)),
    mesh=plsc.VectorSubcoreMesh(core_axis_name="c", subcore_axis_name="s"),
    scratch_types=_SC_SCRATCH,
    compiler_params=pltpu.CompilerParams(needs_layout_passes=False),
)(_sc_body)


def _fin_body(d_ref, n_ref, o_ref):
    o_ref[0, 0] = (jnp.sum(jnp.log(d_ref[...])) -
                   jnp.sum(jnp.log(n_ref[...]))) / float(B * T)


def _finalize(dout, nout):
    return pl.pallas_call(
        _fin_body,
        out_shape=jax.ShapeDtypeStruct((1, 1), jnp.float32),
        in_specs=[pl.BlockSpec(memory_space=pltpu.VMEM),
                  pl.BlockSpec(memory_space=pltpu.VMEM)],
        out_specs=pl.BlockSpec(memory_space=pltpu.SMEM),
    )(dout, nout)[0, 0]


def _pad1d(v, n, dtype):
    out = jnp.zeros((n,), dtype)
    return out.at[: v.shape[0]].set(v.astype(dtype))


def _pair_pack_bf16(tp):
    tpr = tp.reshape(-1, 2, L).astype(jnp.bfloat16)
    bits = lax.bitcast_convert_type(tpr, jnp.uint16).astype(jnp.uint32)
    words = bits[:, 0, :] | (bits[:, 1, :] << 16)
    return lax.bitcast_convert_type(words.reshape(-1), jnp.int32)


def kernel(x, den_src, den_dst, den_pdf, den_tprobs, den_init, den_final,
           num_src, num_dst, num_pdf, num_tprobs, num_init, num_final):
    f32 = jnp.float32
    i32 = jnp.int32
    dpk = (den_src.astype(i32) | (den_dst.astype(i32) << 10)
           | (den_pdf.astype(i32) << 21))
    npk = (num_src.astype(i32) | (num_dst.astype(i32) << 10)
           | (num_pdf.astype(i32) << 21))
    dpk = _pad1d(dpk, EP_DEN, i32)
    dtp = _pair_pack_bf16(_pad1d(den_tprobs, EP_DEN, f32))
    npk = _pad1d(npk, EP_NUM, i32)
    ntp = _pair_pack_bf16(_pad1d(num_tprobs, EP_NUM, f32))
    dinit = _pad1d(den_init, SP_DEN, f32)
    dfinal = _pad1d(den_final, SP_DEN, f32)
    ninit = _pad1d(num_init, SP_NUM, f32)
    nfinal = _pad1d(num_final, SP_NUM, f32)

    dout, nout = _sc_fwd(x.astype(f32), dpk, dtp, npk, ntp,
                         dinit, dfinal, ninit, nfinal)
    return _finalize(dout, nout)
